# baseline (speedup 1.0000x reference)
import functools

import numpy as np
import jax
import jax.numpy as jnp
from jax import lax
from jax.experimental import pallas as pl
from jax.experimental.pallas import tpu as pltpu
from jax.experimental.pallas import tpu_sc as plsc

H = 512
W = 512
L = 16
T_PER = 16384
NEAR = 0.1
FAR = 100.0
OCC_TH = 0.5
HW = H * W
P_TOT = 65536
NC = 2
NS = 16
NW = NC * NS
P_PER = P_TOT // NW
P_PER3 = P_TOT // NS
PR1 = int(np.int32(np.uint32(2654435761)))
PR2 = int(np.int32(np.uint32(805459861)))
RES_F = [float(np.floor(16.0 * (1.3819 ** l))) for l in range(L)]
BMIN = (-1.5, -1.5, 0.0)
BMAX = (1.5, 1.5, 8.0)

WB_N = 288


def _sigmoid(x):
    return 1.0 / (1.0 + jnp.exp(-x))


def _bf16r(v):
    u = plsc.bitcast(v, jnp.int32)
    r = u + 32767 + ((u >> 16) & 1)
    return plsc.bitcast(r & (-65536), jnp.float32)


def _prep_body(means, conf, depth, validf, wpre_o, px_o, py_o, pz_o):
    m = means[...]
    mx, my, mz = m[0], m[1], m[2]
    occ = jnp.mean(conf[...], axis=1).reshape(32, 128)
    gate = _sigmoid((occ - OCC_TH) * 10.0)
    trans = jnp.exp(-depth[...] / FAR)
    wpre_o[...] = occ * gate * trans * validf[...]
    px_o[...] = jnp.clip((mx - BMIN[0]) / (BMAX[0] - BMIN[0]), 0.0, 1.0 - 1e-6)
    py_o[...] = jnp.clip((my - BMIN[1]) / (BMAX[1] - BMIN[1]), 0.0, 1.0 - 1e-6)
    pz_o[...] = jnp.clip((mz - BMIN[2]) / (BMAX[2] - BMIN[2]), 0.0, 1.0 - 1e-6)


_prep = pl.pallas_call(
    _prep_body,
    grid=(16,),
    out_shape=(
        jax.ShapeDtypeStruct((512, 128), jnp.float32),
        jax.ShapeDtypeStruct((512, 128), jnp.float32),
        jax.ShapeDtypeStruct((512, 128), jnp.float32),
        jax.ShapeDtypeStruct((512, 128), jnp.float32),
    ),
    in_specs=[
        pl.BlockSpec((3, 32, 128), lambda i: (0, i, 0)),
        pl.BlockSpec((4096, 27), lambda i: (i, 0)),
        pl.BlockSpec((32, 128), lambda i: (i, 0)),
        pl.BlockSpec((32, 128), lambda i: (i, 0)),
    ],
    out_specs=(
        pl.BlockSpec((32, 128), lambda i: (i, 0)),
        pl.BlockSpec((32, 128), lambda i: (i, 0)),
        pl.BlockSpec((32, 128), lambda i: (i, 0)),
        pl.BlockSpec((32, 128), lambda i: (i, 0)),
    ),
)


def _phase2_body(posx_h, posy_h, posz_h, wpre_h, tbl_h, wb_h,
                 vals_o, wsafe_o,
                 posx_v, posy_v, posz_v, wpre_v, tbl_v0, acc_v,
                 vals_v, wb_v, sem0):
    c = lax.axis_index("c")
    s = lax.axis_index("s")
    wid = s * NC + c
    base = wid * P_PER
    pltpu.sync_copy(posx_h.at[pl.ds(base, P_PER)], posx_v)
    pltpu.sync_copy(posy_h.at[pl.ds(base, P_PER)], posy_v)
    pltpu.sync_copy(posz_h.at[pl.ds(base, P_PER)], posz_v)
    pltpu.sync_copy(wpre_h.at[pl.ds(base, P_PER)], wpre_v)
    pltpu.sync_copy(wb_h, wb_v)

    zf = jnp.zeros((16,), jnp.float32)
    ngrp = P_PER // 16

    def zbody(i, _):
        for j in range(8):
            acc_v[j, pl.ds(i * 16, 16)] = zf
        return _

    lax.fori_loop(0, ngrp, zbody, None)

    def bcast(i):
        return plsc.load_gather(wb_v, [jnp.full((16,), i, jnp.int32)])

    pltpu.async_copy(tbl_h.at[pl.ds(0, 2 * T_PER)],
                     tbl_v0.at[pl.ds(0, 2 * T_PER)], sem0)

    def level_body(l, _):
        off = (l % 2) * (2 * T_PER)
        pltpu.make_async_copy(tbl_h.at[pl.ds(0, 2 * T_PER)],
                              tbl_v0.at[pl.ds(off, 2 * T_PER)], sem0).wait()

        @pl.when(l + 1 < L)
        def _prefetch():
            pltpu.async_copy(
                tbl_h.at[pl.ds((l + 1) * (2 * T_PER), 2 * T_PER)],
                tbl_v0.at[pl.ds((2 * T_PER) - off, 2 * T_PER)], sem0)

        res = bcast(264 + l)
        w0 = [bcast(l * 16 + j) for j in range(8)]
        w1 = [bcast(l * 16 + 8 + j) for j in range(8)]

        def group(gi, __):
            o = gi * 16
            x = posx_v[pl.ds(o, 16)] * res
            y = posy_v[pl.ds(o, 16)] * res
            z = posz_v[pl.ds(o, 16)] * res
            xi = x.astype(jnp.int32)
            yi = y.astype(jnp.int32)
            zi = z.astype(jnp.int32)
            wx = x - xi.astype(jnp.float32)
            wy = y - yi.astype(jnp.float32)
            wz = z - zi.astype(jnp.float32)
            hx = (xi, xi + 1)
            hy = (yi * PR1, yi * PR1 + PR1)
            hz = (zi * PR2, zi * PR2 + PR2)
            wxs = (1.0 - wx, wx)
            wys = (1.0 - wy, wy)
            wzs = (1.0 - wz, wz)
            f0 = zf
            f1 = zf
            for dx in (0, 1):
                for dy in (0, 1):
                    hxy = hx[dx] ^ hy[dy]
                    wxy = wxs[dx] * wys[dy]
                    for dz in (0, 1):
                        h = hxy ^ hz[dz]
                        ix = h & (T_PER - 1)
                        ib = ix + ix + off
                        g0 = plsc.load_gather(tbl_v0, [ib])
                        g1 = plsc.load_gather(tbl_v0, [ib + 1])
                        wc = wxy * wzs[dz]
                        f0 = f0 + wc * g0
                        f1 = f1 + wc * g1
            f0 = _bf16r(f0)
            f1 = _bf16r(f1)
            for j in range(8):
                a = acc_v[j, pl.ds(o, 16)]
                acc_v[j, pl.ds(o, 16)] = a + f0 * w0[j] + f1 * w1[j]
            return __

        lax.fori_loop(0, ngrp, group, None)
        return _

    lax.fori_loop(0, L, level_body, None)

    bias = [bcast(256 + j) for j in range(8)]

    def fin(gi, _):
        o = gi * 16
        outs = [acc_v[j, pl.ds(o, 16)] + bias[j] for j in range(8)]
        wgt = _sigmoid(outs[3]) * wpre_v[pl.ds(o, 16)]
        vals_v[0, pl.ds(o, 16)] = wgt
        for j in range(3):
            vals_v[1 + j, pl.ds(o, 16)] = wgt * _sigmoid(outs[j])
        for j in range(8):
            vals_v[4 + j, pl.ds(o, 16)] = wgt * outs[j]
        return _

    lax.fori_loop(0, ngrp, fin, None)

    for r in range(12):
        pltpu.sync_copy(vals_v.at[r], vals_o.at[r, pl.ds(base, P_PER)])
    pltpu.sync_copy(vals_v.at[0], wsafe_o.at[pl.ds(base, P_PER)])


def _make_phase2():
    mesh = plsc.VectorSubcoreMesh(core_axis_name="c", subcore_axis_name="s",
                                  num_cores=NC, num_subcores=NS)
    return pl.kernel(
        _phase2_body,
        out_type=(
            jax.ShapeDtypeStruct((12, P_TOT), jnp.float32),
            jax.ShapeDtypeStruct((P_TOT,), jnp.float32),
        ),
        mesh=mesh,
        compiler_params=pltpu.CompilerParams(needs_layout_passes=False),
        scratch_types=[
            pltpu.VMEM((P_PER,), jnp.float32),
            pltpu.VMEM((P_PER,), jnp.float32),
            pltpu.VMEM((P_PER,), jnp.float32),
            pltpu.VMEM((P_PER,), jnp.float32),
            pltpu.VMEM((4 * T_PER,), jnp.float32),
            pltpu.VMEM((8, P_PER), jnp.float32),
            pltpu.VMEM((12, P_PER), jnp.float32),
            pltpu.VMEM((WB_N,), jnp.float32),
            pltpu.SemaphoreType.DMA,
        ],
    )


def _phase3_body(pix_h, vals_h, out_h,
                 idx_v, src_v, stage_v, sh0, sh1, sh2, sh3, sh4, sh5, semz):
    c = lax.axis_index("c")
    s = lax.axis_index("s")
    shs = [sh0, sh1, sh2, sh3, sh4, sh5]
    seg = HW // NS
    zf = jnp.zeros((16,), jnp.float32)

    cp_pix = pltpu.async_copy(pix_h.at[s], idx_v, semz)

    def zbody(i, _):
        stage_v[pl.ds(i * 16, 16)] = zf
        return _

    lax.fori_loop(0, seg // 16, zbody, None)
    cp_pix.wait()
    zcps = [pltpu.async_copy(stage_v, shs[ch].at[pl.ds(s * seg, seg)], semz)
            for ch in range(6)]
    for cp in zcps:
        cp.wait()
    plsc.subcore_barrier()

    for ch in range(6):
        row = c * 6 + ch
        pltpu.sync_copy(vals_h.at[row, s], src_v)

        def sbody(r, _, _sh=shs[ch]):
            pltpu.sync_copy(src_v.at[r], _sh.at[idx_v.at[r]], add=True)
            return _

        lax.fori_loop(0, P_PER3 // 128, sbody, None)
    plsc.subcore_barrier()

    wcps = [pltpu.async_copy(shs[ch].at[pl.ds(s * seg, seg)],
                             out_h.at[c * 6 + ch, pl.ds(s * seg, seg)], semz)
            for ch in range(6)]
    for cp in wcps:
        cp.wait()


def _make_phase3():
    mesh = plsc.VectorSubcoreMesh(core_axis_name="c", subcore_axis_name="s",
                                  num_cores=NC, num_subcores=NS)
    return pl.kernel(
        _phase3_body,
        out_type=jax.ShapeDtypeStruct((12, HW), jnp.float32),
        mesh=mesh,
        compiler_params=pltpu.CompilerParams(needs_layout_passes=False),
        scratch_types=[
            pltpu.VMEM((P_PER3 // 128, 128), jnp.int32),
            pltpu.VMEM((P_PER3 // 128, 128), jnp.float32),
            pltpu.VMEM((HW // NS,), jnp.float32),
            pltpu.VMEM_SHARED((HW,), jnp.float32),
            pltpu.VMEM_SHARED((HW,), jnp.float32),
            pltpu.VMEM_SHARED((HW,), jnp.float32),
            pltpu.VMEM_SHARED((HW,), jnp.float32),
            pltpu.VMEM_SHARED((HW,), jnp.float32),
            pltpu.VMEM_SHARED((HW,), jnp.float32),
            pltpu.SemaphoreType.DMA,
        ],
    )


def _comp_body(bg, acc, col_o, feat_o):
    a4 = acc[...]
    aw = a4[0]
    a = jnp.clip(aw, 0.0, 1.0)
    denom = aw + 1e-8
    oma = 1.0 - a
    for j in range(3):
        col_o[j] = (a4[1 + j] / denom) * a + oma * bg[j]
    for j in range(8):
        feat_o[j] = (a4[4 + j] / denom) * a


_comp = pl.pallas_call(
    _comp_body,
    grid=(8,),
    out_shape=(
        jax.ShapeDtypeStruct((3, 2048, 128), jnp.float32),
        jax.ShapeDtypeStruct((8, 2048, 128), jnp.float32),
    ),
    in_specs=[
        pl.BlockSpec(memory_space=pltpu.SMEM),
        pl.BlockSpec((12, 256, 128), lambda i: (0, i, 0)),
    ],
    out_specs=(
        pl.BlockSpec((3, 256, 128), lambda i: (0, i, 0)),
        pl.BlockSpec((8, 256, 128), lambda i: (0, i, 0)),
    ),
)


def kernel(means3D, primitive_confidences, feature_table, linear_weights,
           linear_bias, viewmatrix, projmatrix, cam_pos, bg_color):
    P = means3D.shape[0]
    ones = jnp.ones((P, 1), dtype=means3D.dtype)
    hom = jnp.concatenate([means3D, ones], axis=1)
    depth = (hom @ viewmatrix.T)[:, 2]
    p_hom = hom @ projmatrix.T
    wclip = jnp.maximum(p_hom[:, 3:4], 1e-6)
    ndc = p_hom[:, :2] / wclip
    px = ((ndc[:, 0] + 1.0) * W - 1.0) * 0.5
    py = ((ndc[:, 1] + 1.0) * H - 1.0) * 0.5
    valid = ((depth > NEAR) & (depth < FAR) & (px >= 0) & (px < W)
             & (py >= 0) & (py < H))
    pxi = jnp.clip(px.astype(jnp.int32), 0, W - 1)
    pyi = jnp.clip(py.astype(jnp.int32), 0, H - 1)
    pix2 = jnp.where(valid, pyi * W + pxi, 0).reshape(512, 128)

    meansT = means3D.T.reshape(3, 512, 128)
    wpre2, px2, py2, pz2 = _prep(
        meansT, primitive_confidences, depth.reshape(512, 128),
        valid.astype(jnp.float32).reshape(512, 128))

    wb = jnp.concatenate([
        linear_weights.astype(jnp.bfloat16).astype(jnp.float32).reshape(-1),
        linear_bias.astype(jnp.float32),
        jnp.asarray(RES_F, jnp.float32),
        jnp.zeros((WB_N - 256 - 8 - L,), jnp.float32),
    ])
    vals, wsafe = _make_phase2()(
        px2.reshape(P_TOT), py2.reshape(P_TOT), pz2.reshape(P_TOT),
        wpre2.reshape(P_TOT), feature_table.reshape(-1), wb)

    acc = _make_phase3()(pix2.reshape(NS, -1, 128),
                         vals.reshape(12, NS, -1, 128))

    colT, featT = _comp(bg_color, acc.reshape(12, 2048, 128))
    out_color = colT.reshape(3, HW).T.reshape(H, W, 3)
    out_features = featT.reshape(8, HW).T.reshape(H, W, 8)
    return out_color, out_features, wsafe


# speedup vs baseline: 1.1204x; 1.1204x over previous
import functools

import numpy as np
import jax
import jax.numpy as jnp
from jax import lax
from jax.experimental import pallas as pl
from jax.experimental.pallas import tpu as pltpu
from jax.experimental.pallas import tpu_sc as plsc

H = 512
W = 512
L = 16
T_PER = 16384
NEAR = 0.1
FAR = 100.0
OCC_TH = 0.5
HW = H * W
P_TOT = 65536
NC = 2
NS = 16
NW = NC * NS
P_PER = P_TOT // NW
P_PER3 = P_TOT // NS
PR1 = int(np.int32(np.uint32(2654435761)))
PR2 = int(np.int32(np.uint32(805459861)))
RES_F = [float(np.floor(16.0 * (1.3819 ** l))) for l in range(L)]
BMIN = (-1.5, -1.5, 0.0)
BMAX = (1.5, 1.5, 8.0)

WB_N = 288


def _sigmoid(x):
    return 1.0 / (1.0 + jnp.exp(-x))


def _bf16r(v):
    u = plsc.bitcast(v, jnp.int32)
    r = u + 32767 + ((u >> 16) & 1)
    return plsc.bitcast(r & (-65536), jnp.float32)


def _prep_body(means, conf, depth, validf, wpre_o, px_o, py_o, pz_o):
    m = means[...]
    mx, my, mz = m[0], m[1], m[2]
    occ = jnp.mean(conf[...], axis=0)
    gate = _sigmoid((occ - OCC_TH) * 10.0)
    trans = jnp.exp(-depth[...] / FAR)
    wpre_o[...] = occ * gate * trans * validf[...]
    px_o[...] = jnp.clip((mx - BMIN[0]) / (BMAX[0] - BMIN[0]), 0.0, 1.0 - 1e-6)
    py_o[...] = jnp.clip((my - BMIN[1]) / (BMAX[1] - BMIN[1]), 0.0, 1.0 - 1e-6)
    pz_o[...] = jnp.clip((mz - BMIN[2]) / (BMAX[2] - BMIN[2]), 0.0, 1.0 - 1e-6)


_prep = pl.pallas_call(
    _prep_body,
    out_shape=(
        jax.ShapeDtypeStruct((512, 128), jnp.float32),
        jax.ShapeDtypeStruct((512, 128), jnp.float32),
        jax.ShapeDtypeStruct((512, 128), jnp.float32),
        jax.ShapeDtypeStruct((512, 128), jnp.float32),
    ),
    in_specs=[
        pl.BlockSpec(memory_space=pltpu.VMEM),
        pl.BlockSpec(memory_space=pltpu.VMEM),
        pl.BlockSpec(memory_space=pltpu.VMEM),
        pl.BlockSpec(memory_space=pltpu.VMEM),
    ],
)


def _phase2_body(posx_h, posy_h, posz_h, wpre_h, tbl_h, wb_h,
                 vals_o, wsafe_o,
                 posx_v, posy_v, posz_v, wpre_v, tbl_v0, acc_v,
                 vals_v, wb_v, sem0):
    c = lax.axis_index("c")
    s = lax.axis_index("s")
    wid = s * NC + c
    base = wid * P_PER
    pltpu.sync_copy(posx_h.at[pl.ds(base, P_PER)], posx_v)
    pltpu.sync_copy(posy_h.at[pl.ds(base, P_PER)], posy_v)
    pltpu.sync_copy(posz_h.at[pl.ds(base, P_PER)], posz_v)
    pltpu.sync_copy(wpre_h.at[pl.ds(base, P_PER)], wpre_v)
    pltpu.sync_copy(wb_h, wb_v)

    zf = jnp.zeros((16,), jnp.float32)
    ngrp = P_PER // 16

    def zbody(i, _):
        for j in range(8):
            acc_v[j, pl.ds(i * 16, 16)] = zf
        return _

    lax.fori_loop(0, ngrp, zbody, None)

    def bcast(i):
        return plsc.load_gather(wb_v, [jnp.full((16,), i, jnp.int32)])

    pltpu.async_copy(tbl_h.at[pl.ds(0, 2 * T_PER)],
                     tbl_v0.at[pl.ds(0, 2 * T_PER)], sem0)

    def level_body(l, _):
        off = (l % 2) * (2 * T_PER)
        pltpu.make_async_copy(tbl_h.at[pl.ds(0, 2 * T_PER)],
                              tbl_v0.at[pl.ds(off, 2 * T_PER)], sem0).wait()

        @pl.when(l + 1 < L)
        def _prefetch():
            pltpu.async_copy(
                tbl_h.at[pl.ds((l + 1) * (2 * T_PER), 2 * T_PER)],
                tbl_v0.at[pl.ds((2 * T_PER) - off, 2 * T_PER)], sem0)

        res = bcast(264 + l)
        w0 = [bcast(l * 16 + j) for j in range(8)]
        w1 = [bcast(l * 16 + 8 + j) for j in range(8)]

        def group(gi, __):
            o = gi * 16
            x = posx_v[pl.ds(o, 16)] * res
            y = posy_v[pl.ds(o, 16)] * res
            z = posz_v[pl.ds(o, 16)] * res
            xi = x.astype(jnp.int32)
            yi = y.astype(jnp.int32)
            zi = z.astype(jnp.int32)
            wx = x - xi.astype(jnp.float32)
            wy = y - yi.astype(jnp.float32)
            wz = z - zi.astype(jnp.float32)
            hx = (xi, xi + 1)
            hy = (yi * PR1, yi * PR1 + PR1)
            hz = (zi * PR2, zi * PR2 + PR2)
            wxs = (1.0 - wx, wx)
            wys = (1.0 - wy, wy)
            wzs = (1.0 - wz, wz)
            f0 = zf
            f1 = zf
            for dx in (0, 1):
                for dy in (0, 1):
                    hxy = hx[dx] ^ hy[dy]
                    wxy = wxs[dx] * wys[dy]
                    for dz in (0, 1):
                        h = hxy ^ hz[dz]
                        ix = h & (T_PER - 1)
                        ib = ix + ix + off
                        g0 = plsc.load_gather(tbl_v0, [ib])
                        g1 = plsc.load_gather(tbl_v0, [ib + 1])
                        wc = wxy * wzs[dz]
                        f0 = f0 + wc * g0
                        f1 = f1 + wc * g1
            f0 = _bf16r(f0)
            f1 = _bf16r(f1)
            for j in range(8):
                a = acc_v[j, pl.ds(o, 16)]
                acc_v[j, pl.ds(o, 16)] = a + f0 * w0[j] + f1 * w1[j]
            return __

        lax.fori_loop(0, ngrp, group, None)
        return _

    lax.fori_loop(0, L, level_body, None)

    bias = [bcast(256 + j) for j in range(8)]

    def fin(gi, _):
        o = gi * 16
        outs = [acc_v[j, pl.ds(o, 16)] + bias[j] for j in range(8)]
        wgt = _sigmoid(outs[3]) * wpre_v[pl.ds(o, 16)]
        vals_v[0, pl.ds(o, 16)] = wgt
        for j in range(3):
            vals_v[1 + j, pl.ds(o, 16)] = wgt * _sigmoid(outs[j])
        for j in range(8):
            vals_v[4 + j, pl.ds(o, 16)] = wgt * outs[j]
        return _

    lax.fori_loop(0, ngrp, fin, None)

    for r in range(12):
        pltpu.sync_copy(vals_v.at[r], vals_o.at[r, pl.ds(base, P_PER)])
    pltpu.sync_copy(vals_v.at[0], wsafe_o.at[pl.ds(base, P_PER)])


def _make_phase2():
    mesh = plsc.VectorSubcoreMesh(core_axis_name="c", subcore_axis_name="s",
                                  num_cores=NC, num_subcores=NS)
    return pl.kernel(
        _phase2_body,
        out_type=(
            jax.ShapeDtypeStruct((12, P_TOT), jnp.float32),
            jax.ShapeDtypeStruct((P_TOT,), jnp.float32),
        ),
        mesh=mesh,
        compiler_params=pltpu.CompilerParams(needs_layout_passes=False),
        scratch_types=[
            pltpu.VMEM((P_PER,), jnp.float32),
            pltpu.VMEM((P_PER,), jnp.float32),
            pltpu.VMEM((P_PER,), jnp.float32),
            pltpu.VMEM((P_PER,), jnp.float32),
            pltpu.VMEM((4 * T_PER,), jnp.float32),
            pltpu.VMEM((8, P_PER), jnp.float32),
            pltpu.VMEM((12, P_PER), jnp.float32),
            pltpu.VMEM((WB_N,), jnp.float32),
            pltpu.SemaphoreType.DMA,
        ],
    )


def _phase3_body(pix_h, vals_h, out_h,
                 idx_v, src_v, stage_v, sh0, sh1, sh2, sh3, sh4, sh5, semz):
    c = lax.axis_index("c")
    s = lax.axis_index("s")
    shs = [sh0, sh1, sh2, sh3, sh4, sh5]
    seg = HW // NS
    zf = jnp.zeros((16,), jnp.float32)

    cp_pix = pltpu.async_copy(pix_h.at[s], idx_v, semz)

    def zbody(i, _):
        stage_v[pl.ds(i * 16, 16)] = zf
        return _

    lax.fori_loop(0, seg // 16, zbody, None)
    cp_pix.wait()
    zcps = [pltpu.async_copy(stage_v, shs[ch].at[pl.ds(s * seg, seg)], semz)
            for ch in range(6)]
    for cp in zcps:
        cp.wait()
    plsc.subcore_barrier()

    for ch in range(6):
        row = c * 6 + ch
        pltpu.sync_copy(vals_h.at[row, s], src_v)

        def sbody(r, _, _sh=shs[ch]):
            pltpu.sync_copy(src_v.at[r], _sh.at[idx_v.at[r]], add=True)
            return _

        lax.fori_loop(0, P_PER3 // 128, sbody, None)
    plsc.subcore_barrier()

    wcps = [pltpu.async_copy(shs[ch].at[pl.ds(s * seg, seg)],
                             out_h.at[c * 6 + ch, pl.ds(s * seg, seg)], semz)
            for ch in range(6)]
    for cp in wcps:
        cp.wait()


def _make_phase3():
    mesh = plsc.VectorSubcoreMesh(core_axis_name="c", subcore_axis_name="s",
                                  num_cores=NC, num_subcores=NS)
    return pl.kernel(
        _phase3_body,
        out_type=jax.ShapeDtypeStruct((12, HW), jnp.float32),
        mesh=mesh,
        compiler_params=pltpu.CompilerParams(needs_layout_passes=False),
        scratch_types=[
            pltpu.VMEM((P_PER3 // 128, 128), jnp.int32),
            pltpu.VMEM((P_PER3 // 128, 128), jnp.float32),
            pltpu.VMEM((HW // NS,), jnp.float32),
            pltpu.VMEM_SHARED((HW,), jnp.float32),
            pltpu.VMEM_SHARED((HW,), jnp.float32),
            pltpu.VMEM_SHARED((HW,), jnp.float32),
            pltpu.VMEM_SHARED((HW,), jnp.float32),
            pltpu.VMEM_SHARED((HW,), jnp.float32),
            pltpu.VMEM_SHARED((HW,), jnp.float32),
            pltpu.SemaphoreType.DMA,
        ],
    )


def _comp_body(bg, acc, col_o, feat_o):
    a4 = acc[...]
    aw = a4[0]
    a = jnp.clip(aw, 0.0, 1.0)
    denom = aw + 1e-8
    oma = 1.0 - a
    for j in range(3):
        col_o[j] = (a4[1 + j] / denom) * a + oma * bg[j]
    for j in range(8):
        feat_o[j] = (a4[4 + j] / denom) * a


_comp = pl.pallas_call(
    _comp_body,
    grid=(8,),
    out_shape=(
        jax.ShapeDtypeStruct((3, 2048, 128), jnp.float32),
        jax.ShapeDtypeStruct((8, 2048, 128), jnp.float32),
    ),
    in_specs=[
        pl.BlockSpec(memory_space=pltpu.SMEM),
        pl.BlockSpec((12, 256, 128), lambda i: (0, i, 0)),
    ],
    out_specs=(
        pl.BlockSpec((3, 256, 128), lambda i: (0, i, 0)),
        pl.BlockSpec((8, 256, 128), lambda i: (0, i, 0)),
    ),
)


def kernel(means3D, primitive_confidences, feature_table, linear_weights,
           linear_bias, viewmatrix, projmatrix, cam_pos, bg_color):
    P = means3D.shape[0]
    ones = jnp.ones((P, 1), dtype=means3D.dtype)
    hom = jnp.concatenate([means3D, ones], axis=1)
    depth = (hom @ viewmatrix.T)[:, 2]
    p_hom = hom @ projmatrix.T
    wclip = jnp.maximum(p_hom[:, 3:4], 1e-6)
    ndc = p_hom[:, :2] / wclip
    px = ((ndc[:, 0] + 1.0) * W - 1.0) * 0.5
    py = ((ndc[:, 1] + 1.0) * H - 1.0) * 0.5
    valid = ((depth > NEAR) & (depth < FAR) & (px >= 0) & (px < W)
             & (py >= 0) & (py < H))
    pxi = jnp.clip(px.astype(jnp.int32), 0, W - 1)
    pyi = jnp.clip(py.astype(jnp.int32), 0, H - 1)
    pix2 = jnp.where(valid, pyi * W + pxi, 0).reshape(512, 128)

    meansT = means3D.T.reshape(3, 512, 128)
    confT = primitive_confidences.T.reshape(-1, 512, 128)
    wpre2, px2, py2, pz2 = _prep(
        meansT, confT, depth.reshape(512, 128),
        valid.astype(jnp.float32).reshape(512, 128))

    wb = jnp.concatenate([
        linear_weights.astype(jnp.bfloat16).astype(jnp.float32).reshape(-1),
        linear_bias.astype(jnp.float32),
        jnp.asarray(RES_F, jnp.float32),
        jnp.zeros((WB_N - 256 - 8 - L,), jnp.float32),
    ])
    vals, wsafe = _make_phase2()(
        px2.reshape(P_TOT), py2.reshape(P_TOT), pz2.reshape(P_TOT),
        wpre2.reshape(P_TOT), feature_table.reshape(-1), wb)

    acc = _make_phase3()(pix2.reshape(NS, -1, 128),
                         vals.reshape(12, NS, -1, 128))

    colT, featT = _comp(bg_color, acc.reshape(12, 2048, 128))
    out_color = colT.reshape(3, HW).T.reshape(H, W, 3)
    out_features = featT.reshape(8, HW).T.reshape(H, W, 8)
    return out_color, out_features, wsafe
